# baseline (speedup 1.0000x reference)
import functools

import jax
import jax.numpy as jnp
from jax import lax
from jax.experimental import pallas as pl
from jax.experimental.pallas import tpu as pltpu
from jax.experimental.pallas import tpu_sc as plsc

N_FIELDS = 26
VOCAB = 100000
EMB = 6
N_NUMERIC = 13
BATCH = 4096
D_IN = N_FIELDS * EMB + N_NUMERIC
L1 = 1024
L2 = 206

NC, NS = 2, 16
NW = NC * NS
NIDX = BATCH * N_FIELDS
B_PER_W = NIDX // NW


def _sc_gather(flat_tables, gidx):
    mesh = plsc.VectorSubcoreMesh(core_axis_name="c", subcore_axis_name="s")

    @functools.partial(
        pl.kernel,
        mesh=mesh,
        compiler_params=pltpu.CompilerParams(use_tc_tiling_on_sc=False),
        out_type=jax.ShapeDtypeStruct((NIDX, EMB), jnp.float32),
        scratch_types=[
            pltpu.VMEM((B_PER_W,), jnp.int32),
            pltpu.VMEM((B_PER_W, EMB), jnp.float32),
            pltpu.SemaphoreType.DMA,
        ],
    )
    def k(table_hbm, idx_hbm, out_hbm, idx_v, rows_v, sem):
        wid = lax.axis_index("s") * NC + lax.axis_index("c")
        base = wid * B_PER_W
        pltpu.sync_copy(idx_hbm.at[pl.ds(base, B_PER_W)], idx_v)
        pltpu.async_copy(table_hbm.at[idx_v], rows_v, sem).wait()
        pltpu.sync_copy(rows_v, out_hbm.at[pl.ds(base, B_PER_W)])

    return k(flat_tables, gidx)


def _mlp_body(f_ref, w1_ref, b1_ref, w2_ref, b2_ref, w3_ref, b3_ref,
              h2_ref, out_ref):
    f = f_ref[...]
    h1 = jnp.maximum(
        jnp.dot(f, w1_ref[...], preferred_element_type=jnp.float32)
        + b1_ref[...], 0.0)
    h2 = jnp.maximum(
        jnp.dot(h1, w2_ref[...], preferred_element_type=jnp.float32)
        + b2_ref[...], 0.0)
    h2_ref[...] = h2
    z = jnp.dot(h2, w3_ref[...], preferred_element_type=jnp.float32) + b3_ref[...]
    out_ref[...] = jax.nn.sigmoid(z)


def _mlp(feats, W1, b1, W2, b2, W3, b3):
    BB = 512
    grid = (BATCH // BB,)
    h2, out = pl.pallas_call(
        _mlp_body,
        grid=grid,
        in_specs=[
            pl.BlockSpec((BB, D_IN), lambda i: (i, 0)),
            pl.BlockSpec((D_IN, L1), lambda i: (0, 0)),
            pl.BlockSpec((1, L1), lambda i: (0, 0)),
            pl.BlockSpec((L1, L2), lambda i: (0, 0)),
            pl.BlockSpec((1, L2), lambda i: (0, 0)),
            pl.BlockSpec((L2, 1), lambda i: (0, 0)),
            pl.BlockSpec((1, 1), lambda i: (0, 0)),
        ],
        out_specs=[
            pl.BlockSpec((BB, L2), lambda i: (i, 0)),
            pl.BlockSpec((BB, 1), lambda i: (i, 0)),
        ],
        out_shape=[
            jax.ShapeDtypeStruct((BATCH, L2), jnp.float32),
            jax.ShapeDtypeStruct((BATCH, 1), jnp.float32),
        ],
    )(feats, W1, b1.reshape(1, L1), W2, b2.reshape(1, L2), W3,
      b3.reshape(1, 1))
    return h2, out


def kernel(X, tables, W1, b1, W2, b2, W3, b3):
    idx = X[:, :N_FIELDS].astype(jnp.int32)
    gidx = (idx + jnp.arange(N_FIELDS, dtype=jnp.int32) * VOCAB).reshape(-1)
    flat_tables = tables.reshape(N_FIELDS * VOCAB, EMB)
    rows = _sc_gather(flat_tables, gidx)
    embeds_flat = rows.reshape(BATCH, N_FIELDS * EMB)
    feats = jnp.concatenate([embeds_flat, X[:, N_FIELDS:]], axis=1)
    h2, out = _mlp(feats, W1, b1, W2, b2, W3, b3)
    return (embeds_flat, h2, out)


# speedup vs baseline: 1.4357x; 1.4357x over previous
import functools

import jax
import jax.numpy as jnp
from jax import lax
from jax.experimental import pallas as pl
from jax.experimental.pallas import tpu as pltpu
from jax.experimental.pallas import tpu_sc as plsc

N_FIELDS = 26
VOCAB = 100000
EMB = 6
N_NUMERIC = 13
BATCH = 4096
D_IN = N_FIELDS * EMB + N_NUMERIC
L1 = 1024
L2 = 206

NC, NS = 2, 16
NW = NC * NS
NIDX = BATCH * N_FIELDS
B_PER_W = NIDX // NW


NELEM = NIDX * EMB
E_PER_W = NELEM // NW


def _sc_gather(t1d, gidx6):
    mesh = plsc.VectorSubcoreMesh(core_axis_name="c", subcore_axis_name="s")

    @functools.partial(
        pl.kernel,
        mesh=mesh,
        compiler_params=pltpu.CompilerParams(use_tc_tiling_on_sc=False),
        out_type=jax.ShapeDtypeStruct((NELEM,), jnp.float32),
        scratch_types=[
            pltpu.VMEM((E_PER_W,), jnp.int32),
            pltpu.VMEM((E_PER_W,), jnp.float32),
            pltpu.SemaphoreType.DMA,
        ],
    )
    def k(table_hbm, idx_hbm, out_hbm, idx_v, vals_v, sem):
        wid = lax.axis_index("s") * NC + lax.axis_index("c")
        base = wid * E_PER_W
        pltpu.sync_copy(idx_hbm.at[pl.ds(base, E_PER_W)], idx_v)
        pltpu.async_copy(table_hbm.at[idx_v], vals_v, sem).wait()
        pltpu.sync_copy(vals_v, out_hbm.at[pl.ds(base, E_PER_W)])

    return k(t1d, gidx6)


def _mlp_body(f_ref, w1_ref, b1_ref, w2_ref, b2_ref, w3_ref, b3_ref,
              h2_ref, out_ref):
    f = f_ref[...]
    h1 = jnp.maximum(
        jnp.dot(f, w1_ref[...], preferred_element_type=jnp.float32)
        + b1_ref[...], 0.0)
    h2 = jnp.maximum(
        jnp.dot(h1, w2_ref[...], preferred_element_type=jnp.float32)
        + b2_ref[...], 0.0)
    h2_ref[...] = h2
    z = jnp.dot(h2, w3_ref[...], preferred_element_type=jnp.float32) + b3_ref[...]
    out_ref[...] = jax.nn.sigmoid(z)


def _mlp(feats, W1, b1, W2, b2, W3, b3):
    BB = 512
    grid = (BATCH // BB,)
    h2, out = pl.pallas_call(
        _mlp_body,
        grid=grid,
        in_specs=[
            pl.BlockSpec((BB, D_IN), lambda i: (i, 0)),
            pl.BlockSpec((D_IN, L1), lambda i: (0, 0)),
            pl.BlockSpec((1, L1), lambda i: (0, 0)),
            pl.BlockSpec((L1, L2), lambda i: (0, 0)),
            pl.BlockSpec((1, L2), lambda i: (0, 0)),
            pl.BlockSpec((L2, 1), lambda i: (0, 0)),
            pl.BlockSpec((1, 1), lambda i: (0, 0)),
        ],
        out_specs=[
            pl.BlockSpec((BB, L2), lambda i: (i, 0)),
            pl.BlockSpec((BB, 1), lambda i: (i, 0)),
        ],
        out_shape=[
            jax.ShapeDtypeStruct((BATCH, L2), jnp.float32),
            jax.ShapeDtypeStruct((BATCH, 1), jnp.float32),
        ],
    )(feats, W1, b1.reshape(1, L1), W2, b2.reshape(1, L2), W3,
      b3.reshape(1, 1))
    return h2, out


def kernel(X, tables, W1, b1, W2, b2, W3, b3):
    idx = X[:, :N_FIELDS].astype(jnp.int32)
    gidx = (idx + jnp.arange(N_FIELDS, dtype=jnp.int32) * VOCAB).reshape(-1)
    gidx6 = (gidx[:, None] * EMB + jnp.arange(EMB, dtype=jnp.int32)).reshape(-1)
    t1d = tables.reshape(-1)
    vals = _sc_gather(t1d, gidx6)
    embeds_flat = vals.reshape(BATCH, N_FIELDS * EMB)
    feats = jnp.concatenate([embeds_flat, X[:, N_FIELDS:]], axis=1)
    h2, out = _mlp(feats, W1, b1, W2, b2, W3, b3)
    return (embeds_flat, h2, out)
